# baseline (speedup 1.0000x reference)
import functools

import jax
import jax.numpy as jnp
from jax import lax
from jax.experimental import pallas as pl
from jax.experimental.pallas import tpu as pltpu
from jax.experimental.pallas import tpu_sc as plsc

_CHUNK = 128


def _prep_body(wsh_ref, wh_ref, bsh_ref, bh_ref, a2_ref, bc2_ref):
    a = lax.dot_general(
        wsh_ref[...], wh_ref[...], (((0,), (1,)), ((), ())),
        preferred_element_type=jnp.float32)
    bc = lax.dot_general(
        bsh_ref[...], wh_ref[...], (((1,), (1,)), ((), ())),
        preferred_element_type=jnp.float32) + bh_ref[...]
    za = jnp.zeros_like(a)
    a2_ref[...] = jnp.concatenate(
        [jnp.concatenate([a, za], axis=1), jnp.concatenate([za, a], axis=1)],
        axis=0)
    bc2_ref[...] = jnp.concatenate([bc, bc], axis=1)


def _prep(W_sh, W_h, b_sh2, b_h2):
    S, E = W_sh.shape
    H = W_h.shape[0]
    return pl.pallas_call(
        _prep_body,
        out_shape=(
            jax.ShapeDtypeStruct((2 * E, 2 * H), jnp.float32),
            jax.ShapeDtypeStruct((1, 2 * H), jnp.float32),
        ),
    )(W_sh, W_h, b_sh2, b_h2)


def _vocab_body(x_ref, a2_ref, bc2_ref, o_ref):
    x = x_ref[...]
    half = x.shape[1] // 2
    xb = jnp.concatenate([x[:, :half], x[:, half:]], axis=0)
    z = lax.dot_general(xb, a2_ref[...], (((0,), (0,)), ((), ())),
                        preferred_element_type=jnp.float32)
    o_ref[...] = jnp.tanh(z + bc2_ref[...])


def _vocab_transform(emb_t, a2, bc2, cb):
    E, V = emb_t.shape
    H2 = a2.shape[1]
    nblk = (V + cb - 1) // cb
    return pl.pallas_call(
        _vocab_body,
        grid=(nblk,),
        in_specs=[
            pl.BlockSpec((E, cb), lambda j: (0, j)),
            pl.BlockSpec((2 * E, H2), lambda j: (0, 0)),
            pl.BlockSpec((1, H2), lambda j: (0, 0)),
        ],
        out_specs=pl.BlockSpec((cb // 2, H2), lambda j: (j, 0)),
        out_shape=jax.ShapeDtypeStruct((nblk * cb // 2, H2), jnp.float32),
        compiler_params=pltpu.CompilerParams(
            fuse_transposed_lhs_in_matmul=True),
    )(emb_t, a2, bc2)


def _sc_gather_segsum(ytable, idx2d, L):
    n_chunks, _ = idx2d.shape
    V, E = ytable.shape
    nv = E // 16
    info = plsc.get_sparse_core_info()
    nw = info.num_cores * info.num_subcores
    cpw = n_chunks // nw
    bpw = cpw * _CHUNK // L
    assert cpw * nw == n_chunks and bpw * L == cpw * _CHUNK
    nbuf = 8
    g = 6
    assert cpw % nbuf == 0

    mesh = plsc.VectorSubcoreMesh(core_axis_name="c", subcore_axis_name="s")

    @functools.partial(
        pl.kernel,
        mesh=mesh,
        out_type=jax.ShapeDtypeStruct((nw * bpw * E,), jnp.float32),
        scratch_types=[
            pltpu.VMEM((cpw, _CHUNK), jnp.int32),
            [pltpu.VMEM((_CHUNK, E), jnp.float32) for _ in range(nbuf)],
            pltpu.VMEM((bpw * E,), jnp.float32),
            [pltpu.SemaphoreType.DMA for _ in range(nbuf)],
        ],
        compiler_params=pltpu.CompilerParams(use_tc_tiling_on_sc=False),
    )
    def segsum_kernel(tab_hbm, idx_hbm, out_hbm, idx_v, rows, acc, gsem):
        wid = lax.axis_index("s") * info.num_cores + lax.axis_index("c")
        cbase = wid * cpw
        pltpu.sync_copy(idx_hbm.at[pl.ds(cbase, cpw)], idx_v)

        def zero(i, carry):
            acc[pl.ds(i * 16, 16)] = jnp.zeros((16,), jnp.float32)
            return carry

        lax.fori_loop(0, bpw * E // 16, zero, 0)

        for b in range(g):
            pltpu.async_copy(tab_hbm.at[idx_v.at[b]], rows[b], gsem[b])

        def accum(buf, lo, hi, lb):
            def row(i, sums):
                return tuple(
                    sums[k] + buf[i, pl.ds(16 * k, 16)] for k in range(nv))

            sums = plsc.parallel_loop(
                lo, hi, 1, unroll=4,
                carry=tuple(jnp.zeros((16,), jnp.float32) for _ in range(nv))
            )(row)

            @pl.when(lo < hi)
            def _():
                for k in range(nv):
                    plsc.addupdate(
                        acc.at[pl.ds(lb * E + 16 * k, 16)], sums[k])

        def outer(jo, carry):
            for b in range(nbuf):
                j = jo * nbuf + b
                pltpu.make_async_copy(
                    tab_hbm.at[idx_v.at[j]], rows[b], gsem[b]).wait()
                u0 = j * _CHUNK
                lb0 = u0 // L
                split = jnp.minimum((lb0 + 1) * L - u0, _CHUNK)
                accum(rows[b], 0, split, lb0)
                accum(rows[b], split, _CHUNK, lb0 + 1)
                jn = j + g
                bn = (b + g) % nbuf

                @pl.when(jn < cpw)
                def _():
                    pltpu.async_copy(
                        tab_hbm.at[idx_v.at[jn]], rows[bn], gsem[bn])
            return carry

        lax.fori_loop(0, cpw // nbuf, outer, 0)
        pltpu.sync_copy(acc, out_hbm.at[pl.ds(wid * bpw * E, bpw * E)])

    return segsum_kernel(ytable, idx2d)


def _head_body(L, s_ref, wo_ref, bo_ref, out_ref):
    zm = s_ref[...] * (1.0 / L)
    logits = lax.dot_general(
        zm, wo_ref[...], (((1,), (1,)), ((), ())),
        preferred_element_type=jnp.float32) + bo_ref[...]
    m = jnp.max(logits, axis=1, keepdims=True)
    e = jnp.exp(logits - m)
    out_ref[...] = logits - m - jnp.log(jnp.sum(e, axis=1, keepdims=True))


def _head(sums, W_o, b_o2, L):
    B, H = sums.shape
    O = W_o.shape[0]
    return pl.pallas_call(
        functools.partial(_head_body, L),
        out_shape=jax.ShapeDtypeStruct((B, O), jnp.float32),
    )(sums, W_o, b_o2)


def kernel(sequence, task_id, emb0, W_sh, b_sh, W_h, b_h, W_o, b_o):
    B, L = sequence.shape
    V, E = emb0.shape
    H = W_h.shape[0]
    a2, bc2 = _prep(W_sh, W_h, b_sh.reshape(1, -1), b_h.reshape(1, -1))
    cb = 8192
    ypacked = _vocab_transform(emb0.T, a2, bc2, cb=cb)
    ytable = ypacked.reshape(2 * ypacked.shape[0], H)
    seq32 = sequence.astype(jnp.int32)
    hb = cb // 2
    u = seq32 % cb
    fidx = 2 * (hb * (seq32 // cb) + u % hb) + u // hb
    idx2d = jnp.reshape(fidx, (B * L // _CHUNK, _CHUNK))
    sums = _sc_gather_segsum(ytable, idx2d, L).reshape(B, H)
    return _head(sums, W_o, b_o.reshape(1, -1), L)


# speedup vs baseline: 1.0857x; 1.0857x over previous
import functools

import jax
import jax.numpy as jnp
from jax import lax
from jax.experimental import pallas as pl
from jax.experimental.pallas import tpu as pltpu
from jax.experimental.pallas import tpu_sc as plsc

_CHUNK = 128


def _prep_body(wsh_ref, wh_ref, bsh_ref, bh_ref, a2_ref, bc2_ref):
    a = lax.dot_general(
        wsh_ref[...], wh_ref[...], (((0,), (1,)), ((), ())),
        preferred_element_type=jnp.float32)
    bc = lax.dot_general(
        bsh_ref[...], wh_ref[...], (((1,), (1,)), ((), ())),
        preferred_element_type=jnp.float32) + bh_ref[...]
    za = jnp.zeros_like(a)
    a2_ref[...] = jnp.concatenate(
        [jnp.concatenate([a, za], axis=1), jnp.concatenate([za, a], axis=1)],
        axis=0)
    bc2_ref[...] = jnp.concatenate([bc, bc], axis=1)


def _prep(W_sh, W_h, b_sh2, b_h2):
    S, E = W_sh.shape
    H = W_h.shape[0]
    return pl.pallas_call(
        _prep_body,
        out_shape=(
            jax.ShapeDtypeStruct((2 * E, 2 * H), jnp.float32),
            jax.ShapeDtypeStruct((1, 2 * H), jnp.float32),
        ),
    )(W_sh, W_h, b_sh2, b_h2)


def _vocab_body(x_ref, a2_ref, bc2_ref, o_ref):
    x = x_ref[...]
    half = x.shape[1] // 2
    xb = jnp.concatenate([x[:, :half], x[:, half:]], axis=0)
    z = lax.dot_general(xb, a2_ref[...], (((0,), (0,)), ((), ())),
                        preferred_element_type=jnp.float32)
    o_ref[...] = jnp.tanh(z + bc2_ref[...])


def _vocab_transform(emb_t, a2, bc2, cb):
    E, V = emb_t.shape
    H2 = a2.shape[1]
    nblk = (V + cb - 1) // cb
    return pl.pallas_call(
        _vocab_body,
        grid=(nblk,),
        in_specs=[
            pl.BlockSpec((E, cb), lambda j: (0, j)),
            pl.BlockSpec((2 * E, H2), lambda j: (0, 0)),
            pl.BlockSpec((1, H2), lambda j: (0, 0)),
        ],
        out_specs=pl.BlockSpec((cb // 2, H2), lambda j: (j, 0)),
        out_shape=jax.ShapeDtypeStruct((nblk * cb // 2, H2), jnp.float32),
        compiler_params=pltpu.CompilerParams(
            fuse_transposed_lhs_in_matmul=True),
    )(emb_t, a2, bc2)


def _sc_gather_segsum(ytable, idx2d, L):
    n_chunks, _ = idx2d.shape
    V, E = ytable.shape
    nv = E // 16
    info = plsc.get_sparse_core_info()
    nw = info.num_cores * info.num_subcores
    cpw = n_chunks // nw
    bpw = cpw * _CHUNK // L
    assert cpw * nw == n_chunks and bpw * L == cpw * _CHUNK
    nbuf = 8
    g = 6
    assert cpw % nbuf == 0

    mesh = plsc.VectorSubcoreMesh(core_axis_name="c", subcore_axis_name="s")

    @functools.partial(
        pl.kernel,
        mesh=mesh,
        out_type=jax.ShapeDtypeStruct((nw * bpw * E,), jnp.float32),
        scratch_types=[
            pltpu.VMEM((cpw, _CHUNK), jnp.int32),
            [pltpu.VMEM((_CHUNK, E), jnp.float32) for _ in range(nbuf)],
            pltpu.VMEM((bpw * E,), jnp.float32),
            [pltpu.SemaphoreType.DMA for _ in range(nbuf)],
        ],
        compiler_params=pltpu.CompilerParams(use_tc_tiling_on_sc=False),
    )
    def segsum_kernel(tab_hbm, idx_hbm, out_hbm, idx_v, rows, acc, gsem):
        wid = lax.axis_index("s") * info.num_cores + lax.axis_index("c")
        cbase = wid * cpw
        pltpu.sync_copy(idx_hbm.at[pl.ds(cbase, cpw)], idx_v)

        def zero(i, carry):
            acc[pl.ds(i * 16, 16)] = jnp.zeros((16,), jnp.float32)
            return carry

        lax.fori_loop(0, bpw * E // 16, zero, 0)

        for b in range(g):
            pltpu.async_copy(tab_hbm.at[idx_v.at[b]], rows[b], gsem[b])

        def accum(buf, lo, hi, lb):
            def row(i, sums):
                return tuple(
                    sums[k] + buf[i, pl.ds(16 * k, 16)] for k in range(nv))

            sums = plsc.parallel_loop(
                lo, hi, 1, unroll=4,
                carry=tuple(jnp.zeros((16,), jnp.float32) for _ in range(nv))
            )(row)

            @pl.when(lo < hi)
            def _():
                for k in range(nv):
                    plsc.addupdate(
                        acc.at[pl.ds(lb * E + 16 * k, 16)], sums[k])

        def outer(jo, carry):
            for b in range(nbuf):
                j = jo * nbuf + b
                pltpu.make_async_copy(
                    tab_hbm.at[idx_v.at[j]], rows[b], gsem[b]).wait()
                u0 = j * _CHUNK
                lb0 = u0 // L
                split = jnp.minimum((lb0 + 1) * L - u0, _CHUNK)
                accum(rows[b], 0, split, lb0)
                accum(rows[b], split, _CHUNK, lb0 + 1)
                jn = j + g
                bn = (b + g) % nbuf

                @pl.when(jn < cpw)
                def _():
                    pltpu.async_copy(
                        tab_hbm.at[idx_v.at[jn]], rows[bn], gsem[bn])
            return carry

        lax.fori_loop(0, cpw // nbuf, outer, 0)
        pltpu.sync_copy(acc, out_hbm.at[pl.ds(wid * bpw * E, bpw * E)])

    return segsum_kernel(ytable, idx2d)


def _head_body(L, s_ref, wo_ref, bo_ref, out_ref):
    zm = s_ref[...] * (1.0 / L)
    logits = lax.dot_general(
        zm, wo_ref[...], (((1,), (1,)), ((), ())),
        preferred_element_type=jnp.float32) + bo_ref[...]
    m = jnp.max(logits, axis=1, keepdims=True)
    e = jnp.exp(logits - m)
    out_ref[...] = logits - m - jnp.log(jnp.sum(e, axis=1, keepdims=True))


def _head(sums, W_o, b_o2, L):
    B, H = sums.shape
    O = W_o.shape[0]
    return pl.pallas_call(
        functools.partial(_head_body, L),
        out_shape=jax.ShapeDtypeStruct((B, O), jnp.float32),
    )(sums, W_o, b_o2)


def kernel(sequence, task_id, emb0, W_sh, b_sh, W_h, b_h, W_o, b_o):
    B, L = sequence.shape
    V, E = emb0.shape
    H = W_h.shape[0]
    a2, bc2 = _prep(W_sh, W_h, b_sh.reshape(1, -1), b_h.reshape(1, -1))
    cb = 16384
    ypacked = _vocab_transform(emb0.T, a2, bc2, cb=cb)
    ytable = ypacked.reshape(2 * ypacked.shape[0], H)
    seq32 = sequence.astype(jnp.int32)
    hb = cb // 2
    u = seq32 % cb
    fidx = 2 * (hb * (seq32 // cb) + u % hb) + u // hb
    idx2d = jnp.reshape(fidx, (B * L // _CHUNK, _CHUNK))
    sums = _sc_gather_segsum(ytable, idx2d, L).reshape(B, H)
    return _head(sums, W_o, b_o.reshape(1, -1), L)


# speedup vs baseline: 1.1054x; 1.0181x over previous
import functools

import jax
import jax.numpy as jnp
from jax import lax
from jax.experimental import pallas as pl
from jax.experimental.pallas import tpu as pltpu
from jax.experimental.pallas import tpu_sc as plsc

_CHUNK = 128


def _prep_body(wsh_ref, wh_ref, bsh_ref, bh_ref, a2_ref, bc2_ref):
    a = lax.dot_general(
        wsh_ref[...], wh_ref[...], (((0,), (1,)), ((), ())),
        preferred_element_type=jnp.float32)
    bc = lax.dot_general(
        bsh_ref[...], wh_ref[...], (((1,), (1,)), ((), ())),
        preferred_element_type=jnp.float32) + bh_ref[...]
    za = jnp.zeros_like(a)
    a2_ref[...] = jnp.concatenate(
        [jnp.concatenate([a, za], axis=1), jnp.concatenate([za, a], axis=1)],
        axis=0)
    bc2_ref[...] = jnp.concatenate([bc, bc], axis=1)


def _prep(W_sh, W_h, b_sh2, b_h2):
    S, E = W_sh.shape
    H = W_h.shape[0]
    return pl.pallas_call(
        _prep_body,
        out_shape=(
            jax.ShapeDtypeStruct((2 * E, 2 * H), jnp.float32),
            jax.ShapeDtypeStruct((1, 2 * H), jnp.float32),
        ),
    )(W_sh, W_h, b_sh2, b_h2)


def _vocab_body(x_ref, a2_ref, bc2_ref, o_ref):
    x = x_ref[...]
    half = x.shape[1] // 2
    xb = jnp.concatenate([x[:, :half], x[:, half:]], axis=0)
    z = lax.dot_general(xb, a2_ref[...], (((0,), (0,)), ((), ())),
                        preferred_element_type=jnp.float32)
    o_ref[...] = jnp.tanh(z + bc2_ref[...])


def _vocab_transform(emb_t, a2, bc2, cb):
    E, V = emb_t.shape
    H2 = a2.shape[1]
    nblk = (V + cb - 1) // cb
    return pl.pallas_call(
        _vocab_body,
        grid=(nblk,),
        in_specs=[
            pl.BlockSpec((E, cb), lambda j: (0, j)),
            pl.BlockSpec((2 * E, H2), lambda j: (0, 0)),
            pl.BlockSpec((1, H2), lambda j: (0, 0)),
        ],
        out_specs=pl.BlockSpec((cb // 2, H2), lambda j: (j, 0)),
        out_shape=jax.ShapeDtypeStruct((nblk * cb // 2, H2), jnp.float32),
        compiler_params=pltpu.CompilerParams(
            fuse_transposed_lhs_in_matmul=True),
    )(emb_t, a2, bc2)


def _sc_gather_segsum(ytable, idx2d, L):
    n_chunks, _ = idx2d.shape
    V, E = ytable.shape
    nv = E // 16
    info = plsc.get_sparse_core_info()
    nw = info.num_cores * info.num_subcores
    cpw = n_chunks // nw
    bpw = cpw * _CHUNK // L
    assert cpw * nw == n_chunks and bpw * L == cpw * _CHUNK
    nbuf = 8
    g = 6
    assert cpw % nbuf == 0

    mesh = plsc.VectorSubcoreMesh(core_axis_name="c", subcore_axis_name="s")

    @functools.partial(
        pl.kernel,
        mesh=mesh,
        out_type=jax.ShapeDtypeStruct((nw * bpw * E,), jnp.float32),
        scratch_types=[
            pltpu.VMEM((cpw, _CHUNK), jnp.int32),
            [pltpu.VMEM((_CHUNK, E), jnp.float32) for _ in range(nbuf)],
            pltpu.VMEM((bpw * E,), jnp.float32),
            [pltpu.SemaphoreType.DMA for _ in range(nbuf)],
        ],
        compiler_params=pltpu.CompilerParams(use_tc_tiling_on_sc=False),
    )
    def segsum_kernel(tab_hbm, idx_hbm, out_hbm, idx_v, rows, acc, gsem):
        wid = lax.axis_index("s") * info.num_cores + lax.axis_index("c")
        cbase = wid * cpw
        pltpu.sync_copy(idx_hbm.at[pl.ds(cbase, cpw)], idx_v)

        def zero(i, carry):
            acc[pl.ds(i * 16, 16)] = jnp.zeros((16,), jnp.float32)
            return carry

        lax.fori_loop(0, bpw * E // 16, zero, 0)

        for b in range(g):
            pltpu.async_copy(tab_hbm.at[idx_v.at[b]], rows[b], gsem[b])

        def accum(buf, lo, hi, lb):
            def row(i, sums):
                return tuple(
                    sums[k] + buf[i, pl.ds(16 * k, 16)] for k in range(nv))

            sums = plsc.parallel_loop(
                lo, hi, 1, unroll=4,
                carry=tuple(jnp.zeros((16,), jnp.float32) for _ in range(nv))
            )(row)

            @pl.when(lo < hi)
            def _():
                for k in range(nv):
                    plsc.addupdate(
                        acc.at[pl.ds(lb * E + 16 * k, 16)], sums[k])

        def outer(jo, carry):
            for b in range(nbuf):
                j = jo * nbuf + b
                pltpu.make_async_copy(
                    tab_hbm.at[idx_v.at[j]], rows[b], gsem[b]).wait()
                u0 = j * _CHUNK
                lb0 = u0 // L
                split = jnp.minimum((lb0 + 1) * L - u0, _CHUNK)
                accum(rows[b], 0, split, lb0)
                accum(rows[b], split, _CHUNK, lb0 + 1)
                jn = j + g
                bn = (b + g) % nbuf

                @pl.when(jn < cpw)
                def _():
                    pltpu.async_copy(
                        tab_hbm.at[idx_v.at[jn]], rows[bn], gsem[bn])
            return carry

        lax.fori_loop(0, cpw // nbuf, outer, 0)
        pltpu.sync_copy(acc, out_hbm.at[pl.ds(wid * bpw * E, bpw * E)])

    return segsum_kernel(ytable, idx2d)


def _head_body(L, s_ref, wo_ref, bo_ref, out_ref):
    zm = s_ref[...] * (1.0 / L)
    logits = lax.dot_general(
        zm, wo_ref[...], (((1,), (1,)), ((), ())),
        preferred_element_type=jnp.float32) + bo_ref[...]
    m = jnp.max(logits, axis=1, keepdims=True)
    e = jnp.exp(logits - m)
    out_ref[...] = logits - m - jnp.log(jnp.sum(e, axis=1, keepdims=True))


def _head(sums, W_o, b_o2, L):
    B, H = sums.shape
    O = W_o.shape[0]
    return pl.pallas_call(
        functools.partial(_head_body, L),
        out_shape=jax.ShapeDtypeStruct((B, O), jnp.float32),
    )(sums, W_o, b_o2)


def kernel(sequence, task_id, emb0, W_sh, b_sh, W_h, b_h, W_o, b_o):
    B, L = sequence.shape
    V, E = emb0.shape
    H = W_h.shape[0]
    a2, bc2 = _prep(W_sh, W_h, b_sh.reshape(1, -1), b_h.reshape(1, -1))
    cb = 32768
    ypacked = _vocab_transform(emb0.T, a2, bc2, cb=cb)
    ytable = ypacked.reshape(2 * ypacked.shape[0], H)
    seq32 = sequence.astype(jnp.int32)
    hb = cb // 2
    u = seq32 % cb
    fidx = 2 * (hb * (seq32 // cb) + u % hb) + u // hb
    idx2d = jnp.reshape(fidx, (B * L // _CHUNK, _CHUNK))
    sums = _sc_gather_segsum(ytable, idx2d, L).reshape(B, H)
    return _head(sums, W_o, b_o.reshape(1, -1), L)
